# Initial kernel scaffold; baseline (speedup 1.0000x reference)
import jax
import jax.numpy as jnp
from jax.experimental import pallas as pl


def kernel(x, tables):
    raise NotImplementedError("write your pallas kernel here")


# speedup vs baseline: 1.2073x; 1.2073x over previous
import functools

import jax
import jax.numpy as jnp
from jax import lax
from jax.experimental import pallas as pl
from jax.experimental.pallas import tpu as pltpu
from jax.experimental.pallas import tpu_sc as plsc

N_WORKERS = 32
IDX_PER_DMA = 128
ROWS_PER_BLOCK = 1024


def _build_gather(total, per_w, F, V, D):
    n_rows = per_w // IDX_PER_DMA
    n_blocks = per_w // ROWS_PER_BLOCK
    dmas_per_block = ROWS_PER_BLOCK // IDX_PER_DMA
    mesh = plsc.VectorSubcoreMesh(core_axis_name="c", subcore_axis_name="s")

    @functools.partial(
        pl.kernel,
        mesh=mesh,
        out_type=jax.ShapeDtypeStruct((total, D), jnp.float32),
        scratch_types=[
            pltpu.VMEM((n_rows, IDX_PER_DMA), jnp.int32),
            pltpu.VMEM((ROWS_PER_BLOCK, D), jnp.float32),
            pltpu.SemaphoreType.DMA,
        ],
        compiler_params=pltpu.CompilerParams(use_tc_tiling_on_sc=False),
    )
    def gather_kernel(x_hbm, tab_hbm, out_hbm, idx_v, rows_v, sem):
        wid = lax.axis_index("s") * 2 + lax.axis_index("c")
        pltpu.sync_copy(x_hbm.at[wid], idx_v)

        def add_offs(j, carry):
            row = idx_v.at[j]
            for kk in range(IDX_PER_DMA // 16):
                pos = j * IDX_PER_DMA + kk * 16 + lax.iota(jnp.int32, 16)
                off = lax.rem(pos, F) * V
                row[pl.ds(kk * 16, 16)] = row[pl.ds(kk * 16, 16)] + off
            return carry

        lax.fori_loop(0, n_rows, add_offs, 0)

        def blk_body(blk, carry):
            copies = [
                pltpu.async_copy(
                    tab_hbm.at[idx_v.at[blk * dmas_per_block + jj]],
                    rows_v.at[pl.ds(jj * IDX_PER_DMA, IDX_PER_DMA)],
                    sem,
                )
                for jj in range(dmas_per_block)
            ]
            for cp in copies:
                cp.wait()
            base = wid * per_w + blk * ROWS_PER_BLOCK
            pltpu.sync_copy(rows_v, out_hbm.at[pl.ds(base, ROWS_PER_BLOCK)])
            return carry

        lax.fori_loop(0, n_blocks, blk_body, 0)

    return gather_kernel


def kernel(x, tables):
    B, F = x.shape
    _, V, D = tables.shape
    total = B * F
    per_w = total // N_WORKERS
    x3 = x.reshape(N_WORKERS, per_w // IDX_PER_DMA, IDX_PER_DMA)
    tab = tables.reshape(F * V, D)
    out = _build_gather(total, per_w, F, V, D)(x3, tab)
    return out.reshape(B, F * D)


# speedup vs baseline: 1.2145x; 1.0060x over previous
import functools

import jax
import jax.numpy as jnp
from jax import lax
from jax.experimental import pallas as pl
from jax.experimental.pallas import tpu as pltpu
from jax.experimental.pallas import tpu_sc as plsc

N_WORKERS = 32
IDX_PER_DMA = 128
NBUF = 8


def _build_gather(total, per_w, F, V, D):
    n_blocks = per_w // IDX_PER_DMA
    mesh = plsc.VectorSubcoreMesh(core_axis_name="c", subcore_axis_name="s")

    @functools.partial(
        pl.kernel,
        mesh=mesh,
        out_type=jax.ShapeDtypeStruct((total, D), jnp.float32),
        scratch_types=[
            pltpu.VMEM((n_blocks, IDX_PER_DMA), jnp.int32),
            pltpu.VMEM((NBUF, IDX_PER_DMA, D), jnp.float32),
            pltpu.SemaphoreType.DMA((NBUF,)),
        ],
        compiler_params=pltpu.CompilerParams(use_tc_tiling_on_sc=False),
    )
    def gather_kernel(x_hbm, tab_hbm, out_hbm, idx_v, rows_v, sems):
        wid = lax.axis_index("s") * 2 + lax.axis_index("c")
        pltpu.sync_copy(x_hbm.at[wid], idx_v)

        def add_offs(j, carry):
            row = idx_v.at[j]
            for kk in range(IDX_PER_DMA // 16):
                pos = j * IDX_PER_DMA + kk * 16 + lax.iota(jnp.int32, 16)
                off = lax.rem(pos, F) * V
                row[pl.ds(kk * 16, 16)] = row[pl.ds(kk * 16, 16)] + off
            return carry

        lax.fori_loop(0, n_blocks, add_offs, 0)

        def fire(blk, b):
            return pltpu.make_async_copy(
                tab_hbm.at[idx_v.at[blk]], rows_v.at[b], sems.at[b]
            )

        for b in range(NBUF):
            fire(b, b).start()

        def blk_body(it, carry):
            for b in range(NBUF):
                blk = it * NBUF + b
                fire(blk, b).wait()
                base = wid * per_w + blk * IDX_PER_DMA
                pltpu.sync_copy(rows_v.at[b], out_hbm.at[pl.ds(base, IDX_PER_DMA)])
                nxt = blk + NBUF

                @pl.when(nxt < n_blocks)
                def _():
                    fire(nxt, b).start()

            return carry

        lax.fori_loop(0, n_blocks // NBUF, blk_body, 0)

    return gather_kernel


def kernel(x, tables):
    B, F = x.shape
    _, V, D = tables.shape
    total = B * F
    per_w = total // N_WORKERS
    x3 = x.reshape(N_WORKERS, per_w // IDX_PER_DMA, IDX_PER_DMA)
    tab = tables.reshape(F * V, D)
    out = _build_gather(total, per_w, F, V, D)(x3, tab)
    return out.reshape(B, F * D)


# speedup vs baseline: 7.7809x; 6.4068x over previous
import functools

import jax
import jax.numpy as jnp
from jax import lax
from jax.experimental import pallas as pl
from jax.experimental.pallas import tpu as pltpu
from jax.experimental.pallas import tpu_sc as plsc

N_WORKERS = 32
CHUNK = 4096


def _build_colgather(B, F, V, D):
    n_pairs = F * D
    pairs_per_w = n_pairs // N_WORKERS
    n_chunks = B // CHUNK
    mesh = plsc.VectorSubcoreMesh(core_axis_name="c", subcore_axis_name="s")

    @functools.partial(
        pl.kernel,
        mesh=mesh,
        out_type=jax.ShapeDtypeStruct((n_pairs, B), jnp.float32),
        scratch_types=[
            pltpu.VMEM((V,), jnp.float32),
            pltpu.VMEM((B,), jnp.int32),
            pltpu.VMEM((CHUNK,), jnp.float32),
        ],
        compiler_params=pltpu.CompilerParams(needs_layout_passes=False),
    )
    def colgather_kernel(xt_hbm, tabt_hbm, out_hbm, col_v, idx_v, res_v):
        wid = lax.axis_index("s") * 2 + lax.axis_index("c")

        def pair_body(j, f_prev):
            g = wid * pairs_per_w + j
            f = g // D
            pltpu.sync_copy(tabt_hbm.at[g], col_v)

            @pl.when(f != f_prev)
            def _():
                pltpu.sync_copy(xt_hbm.at[f], idx_v)

            def chunk_body(c, carry):
                @plsc.parallel_loop(0, CHUNK, step=16, unroll=8)
                def _(i):
                    idx16 = idx_v[pl.ds(c * CHUNK + i, 16)]
                    res_v[pl.ds(i, 16)] = plsc.load_gather(col_v, [idx16])

                pltpu.sync_copy(res_v, out_hbm.at[g, pl.ds(c * CHUNK, CHUNK)])
                return carry

            lax.fori_loop(0, n_chunks, chunk_body, 0)
            return f

        lax.fori_loop(0, pairs_per_w, pair_body, jnp.int32(-1))

    return colgather_kernel


def kernel(x, tables):
    B, F = x.shape
    _, V, D = tables.shape
    tabt = tables.transpose(0, 2, 1).reshape(F * D, V)
    xt = x.T
    out = _build_colgather(B, F, V, D)(xt, tabt)
    return out.T


# speedup vs baseline: 7.9797x; 1.0256x over previous
import functools

import jax
import jax.numpy as jnp
from jax import lax
from jax.experimental import pallas as pl
from jax.experimental.pallas import tpu as pltpu
from jax.experimental.pallas import tpu_sc as plsc

N_WORKERS = 32
CHUNK = 4096


def _build_colgather(B, F, V, D):
    n_pairs = F * D
    pairs_per_w = n_pairs // N_WORKERS
    n_chunks = B // CHUNK
    mesh = plsc.VectorSubcoreMesh(core_axis_name="c", subcore_axis_name="s")

    @functools.partial(
        pl.kernel,
        mesh=mesh,
        out_type=jax.ShapeDtypeStruct((n_pairs, B), jnp.float32),
        scratch_types=[
            pltpu.VMEM((V,), jnp.float32),
            pltpu.VMEM((B,), jnp.int32),
            pltpu.VMEM((2, CHUNK), jnp.float32),
            pltpu.SemaphoreType.DMA,
            pltpu.SemaphoreType.DMA((2,)),
        ],
        compiler_params=pltpu.CompilerParams(needs_layout_passes=False),
    )
    def colgather_kernel(xt_hbm, tabt_hbm, out_hbm, col_v, idx_v, res_v, semc, sems):
        sid = lax.axis_index("s")
        wid = sid * 2 + lax.axis_index("c")

        @pl.when(sid % 2 == 1)
        def _():
            pl.delay(3000)

        def store_cp(g, c, b):
            return pltpu.make_async_copy(
                res_v.at[b], out_hbm.at[g, pl.ds(c * CHUNK, CHUNK)], sems.at[b]
            )

        def pair_body(j, f_prev):
            g = wid * pairs_per_w + j
            f = g // D
            col_cp = pltpu.make_async_copy(tabt_hbm.at[g], col_v, semc)
            col_cp.start()

            @pl.when(f != f_prev)
            def _():
                pltpu.sync_copy(xt_hbm.at[f], idx_v)

            col_cp.wait()
            for c in range(n_chunks):
                b = c % 2

                @pl.when(j * n_chunks + c >= 2)
                def _():
                    store_cp(g, c, b).wait()

                @plsc.parallel_loop(0, CHUNK, step=16, unroll=16)
                def _(i):
                    idx16 = idx_v[pl.ds(c * CHUNK + i, 16)]
                    res_v[b, pl.ds(i, 16)] = plsc.load_gather(col_v, [idx16])

                store_cp(g, c, b).start()
            return f

        f_last = lax.fori_loop(0, pairs_per_w, pair_body, jnp.int32(-1))
        g_last = wid * pairs_per_w + pairs_per_w - 1
        for c in (n_chunks - 2, n_chunks - 1):
            store_cp(g_last, c, c % 2).wait()

    return colgather_kernel


def kernel(x, tables):
    B, F = x.shape
    _, V, D = tables.shape
    tabt = tables.transpose(0, 2, 1).reshape(F * D, V)
    xt = x.T
    out = _build_colgather(B, F, V, D)(xt, tabt)
    return out.T
